# Initial kernel scaffold; baseline (speedup 1.0000x reference)
import jax
import jax.numpy as jnp
from jax.experimental import pallas as pl


def kernel(x, edge_index, edge_weight, batch, Wr1, br1, Ws1, Wr2, br2, Ws2, Wr3, br3, Ws3):
    raise NotImplementedError("write your pallas kernel here")


# speedup vs baseline: 5.2284x; 5.2284x over previous
import dataclasses
import functools

import jax
import jax.numpy as jnp
from jax import lax
from jax.experimental import pallas as pl
from jax.experimental.pallas import tpu as pltpu
from jax.experimental.pallas import tpu_sc as plsc

N = 10000
NPAD = 10240
E = 160000
CHUNK = 128
NCHUNKS = E // CHUNK
NTILES = 16
ROWS_PER_TILE = NPAD // NTILES
ZROWS = 128
F32 = jnp.float32


def _segsum_sc(p_parts, src, dst, ew, Dc, col_split):
    mesh = plsc.VectorSubcoreMesh(core_axis_name="c", subcore_axis_name="s")
    n_j = Dc // 16

    def body(*refs):
        if col_split:
            (p0_hbm, p1_hbm, src_hbm, dst_hbm, ew_hbm, o0_hbm, o1_hbm,
             src_v, dst_v, ew_v, rows_v, zbuf, acc) = refs
        else:
            (p0_hbm, src_hbm, dst_hbm, ew_hbm, o0_hbm, o1_hbm,
             src_v, dst_v, ew_v, rows_v, zbuf, acc) = refs
        c = lax.axis_index("c")
        s = lax.axis_index("s")

        @pl.loop(0, ZROWS)
        def _(i):
            for j in range(n_j):
                zbuf[i, pl.ds(j * 16, 16)] = jnp.zeros((16,), F32)
        for k in range(ROWS_PER_TILE // ZROWS):
            pltpu.sync_copy(
                zbuf, acc.at[pl.ds(s * ROWS_PER_TILE + k * ZROWS, ZROWS)])
        plsc.subcore_barrier()

        def chunk_work(table_hbm, start, step):
            @pl.loop(start, NCHUNKS, step=step)
            def _(g):
                off = g * CHUNK
                pltpu.sync_copy(src_hbm.at[pl.ds(off, CHUNK)], src_v)
                pltpu.sync_copy(dst_hbm.at[pl.ds(off, CHUNK)], dst_v)
                pltpu.sync_copy(ew_hbm.at[pl.ds(off, CHUNK)], ew_v)
                pltpu.sync_copy(table_hbm.at[src_v], rows_v)

                @pl.loop(0, CHUNK)
                def _(e):
                    w = plsc.load_gather(ew_v, [jnp.full((16,), e, jnp.int32)])
                    for j in range(n_j):
                        sl = (e, pl.ds(j * 16, 16))
                        rows_v[sl] = rows_v[sl] * w

                pltpu.sync_copy(rows_v, acc.at[dst_v], add=True)

        if col_split:
            pl.when(c == 0)(lambda: chunk_work(p0_hbm, s, NTILES))
            pl.when(c == 1)(lambda: chunk_work(p1_hbm, s, NTILES))
        else:
            chunk_work(p0_hbm, c * NTILES + s, 2 * NTILES)

        plsc.subcore_barrier()
        rsl = pl.ds(s * ROWS_PER_TILE, ROWS_PER_TILE)
        pl.when(c == 0)(lambda: pltpu.sync_copy(acc.at[rsl], o0_hbm.at[rsl]))
        pl.when(c == 1)(lambda: pltpu.sync_copy(acc.at[rsl], o1_hbm.at[rsl]))

    out_t = (jax.ShapeDtypeStruct((NPAD, Dc), F32),
             jax.ShapeDtypeStruct((NPAD, Dc), F32))
    cp = pltpu.CompilerParams()
    if "needs_layout_passes" in pltpu.CompilerParams.__dataclass_fields__:
        cp = dataclasses.replace(cp, needs_layout_passes=False)
    fn = pl.kernel(
        body,
        out_type=out_t,
        mesh=mesh,
        compiler_params=cp,
        scratch_types=[
            pltpu.VMEM((CHUNK,), jnp.int32),
            pltpu.VMEM((CHUNK,), jnp.int32),
            pltpu.VMEM((CHUNK,), F32),
            pltpu.VMEM((CHUNK, Dc), F32),
            pltpu.VMEM((ZROWS, Dc), F32),
            pltpu.VMEM_SHARED((NPAD, Dc), F32),
        ],
    )
    return fn(*p_parts, src, dst, ew)


BN = 400
GRID = N // BN


def _bs(shape, im):
    return pl.BlockSpec(shape, im)


def _row(i):
    return (i, 0)


def _rep(i):
    return (0, 0)


def _tc_project1(x, Wr, Ws, b):
    D = Wr.shape[1]
    Dh = D // 2
    K = x.shape[1]

    def body(x_ref, wr_ref, ws_ref, b_ref, p0_ref, p1_ref, r_ref):
        xb = x_ref[...]
        p = jnp.dot(xb, wr_ref[...], preferred_element_type=F32)
        p0_ref[...] = p[:, :Dh]
        p1_ref[...] = p[:, Dh:]
        r_ref[...] = jnp.dot(xb, ws_ref[...], preferred_element_type=F32) + b_ref[...]

    return pl.pallas_call(
        body,
        grid=(GRID,),
        in_specs=[_bs((BN, K), _row), _bs((K, D), _rep), _bs((K, D), _rep),
                  _bs((1, D), _rep)],
        out_specs=[_bs((BN, Dh), _row), _bs((BN, Dh), _row), _bs((BN, D), _row)],
        out_shape=[jax.ShapeDtypeStruct((N, Dh), F32),
                   jax.ShapeDtypeStruct((N, Dh), F32),
                   jax.ShapeDtypeStruct((N, D), F32)],
    )(x, Wr, Ws, b.reshape(1, D))


def _tc_mid(a0, a1, r, Wr, Ws, b):
    Dh_in = a0.shape[1]
    D = Wr.shape[1]

    def body(a0_ref, a1_ref, r_ref, wr_ref, ws_ref, b_ref, p_ref, r2_ref):
        h = jnp.concatenate([a0_ref[...], a1_ref[...]], axis=1) + r_ref[...]
        h = jnp.maximum(h, 0.0)
        p_ref[...] = jnp.dot(h, wr_ref[...], preferred_element_type=F32)
        r2_ref[...] = jnp.dot(h, ws_ref[...], preferred_element_type=F32) + b_ref[...]

    K = 2 * Dh_in
    return pl.pallas_call(
        body,
        grid=(GRID,),
        in_specs=[_bs((BN, Dh_in), _row), _bs((BN, Dh_in), _row),
                  _bs((BN, K), _row), _bs((K, D), _rep), _bs((K, D), _rep),
                  _bs((1, D), _rep)],
        out_specs=[_bs((BN, D), _row), _bs((BN, D), _row)],
        out_shape=[jax.ShapeDtypeStruct((N, D), F32),
                   jax.ShapeDtypeStruct((N, D), F32)],
    )(a0, a1, r, Wr, Ws, b.reshape(1, D))


def _tc_last_h(a0, a1, r, Ws, b):
    D = a0.shape[1]
    Do = Ws.shape[1]

    def body(a0_ref, a1_ref, r_ref, ws_ref, b_ref, h_ref, r3_ref):
        h = jnp.maximum(a0_ref[...] + a1_ref[...] + r_ref[...], 0.0)
        h_ref[...] = h
        r3_ref[...] = jnp.dot(h, ws_ref[...], preferred_element_type=F32) + b_ref[...]

    return pl.pallas_call(
        body,
        grid=(GRID,),
        in_specs=[_bs((BN, D), _row), _bs((BN, D), _row), _bs((BN, D), _row),
                  _bs((D, Do), _rep), _bs((1, Do), _rep)],
        out_specs=[_bs((BN, D), _row), _bs((BN, Do), _row)],
        out_shape=[jax.ShapeDtypeStruct((N, D), F32),
                   jax.ShapeDtypeStruct((N, Do), F32)],
    )(a0, a1, r, Ws, b.reshape(1, Do))


def _tc_final(b0, b1, r3, Wr):
    D = b0.shape[1]
    Do = Wr.shape[1]

    def body(b0_ref, b1_ref, r_ref, wr_ref, o_ref):
        agg = b0_ref[...] + b1_ref[...]
        o_ref[...] = jnp.dot(agg, wr_ref[...], preferred_element_type=F32) + r_ref[...]

    return pl.pallas_call(
        body,
        grid=(GRID,),
        in_specs=[_bs((BN, D), _row), _bs((BN, D), _row), _bs((BN, Do), _row),
                  _bs((D, Do), _rep)],
        out_specs=_bs((BN, Do), _row),
        out_shape=jax.ShapeDtypeStruct((N, Do), F32),
    )(b0, b1, r3, Wr)


def kernel(x, edge_index, edge_weight, batch, Wr1, br1, Ws1,
           Wr2, br2, Ws2, Wr3, br3, Ws3):
    del batch
    src = edge_index[0]
    dst = edge_index[1]

    p0, p1, r1 = _tc_project1(x, Wr1, Ws1, br1)
    a0, a1 = _segsum_sc((p0, p1), src, dst, edge_weight, 128, col_split=True)

    p2, r2 = _tc_mid(a0, a1, r1, Wr2, Ws2, br2)
    a0, a1 = _segsum_sc((p2,), src, dst, edge_weight, 128, col_split=False)

    h2, r3 = _tc_last_h(a0, a1, r2, Ws3, br3)
    b0, b1 = _segsum_sc((h2,), src, dst, edge_weight, 128, col_split=False)

    return _tc_final(b0, b1, r3, Wr3)
